# baseline (speedup 1.0000x reference)
import functools

import jax
import jax.numpy as jnp
from jax import lax
from jax.experimental import pallas as pl
from jax.experimental.pallas import tpu as pltpu
from jax.experimental.pallas import tpu_sc as plsc

VOCAB = 100000
D = 128
B = 4096
NNEG = 50

NC = 2
NS = 16
NW = NC * NS

BA = B // NW
CH = BA
NBUF = 7
DEPTH = 3

NCH = NNEG
T = 2 + NCH

_mesh = plsc.VectorSubcoreMesh(core_axis_name="c", subcore_axis_name="s")


@functools.partial(
    pl.kernel,
    out_type=(
        jax.ShapeDtypeStruct((B, D), jnp.float32),
        jax.ShapeDtypeStruct((B, D), jnp.float32),
        jax.ShapeDtypeStruct((B * NNEG, D), jnp.float32),
    ),
    mesh=_mesh,
    scratch_types=(
        [pltpu.VMEM((BA,), jnp.int32),
         pltpu.VMEM((BA,), jnp.int32),
         pltpu.VMEM((NNEG, CH), jnp.int32)]
        + [pltpu.VMEM((CH, D), jnp.float32)] * NBUF
        + [pltpu.SemaphoreType.DMA] * (2 * NBUF)
    ),
)
def _gather3(tab, anc, pos, neg, outa, outp, outn, idx_a, idx_p, idx_n, *rest):
    bufs = rest[:NBUF]
    gs = rest[NBUF:2 * NBUF]
    ss = rest[2 * NBUF:]

    wid = lax.axis_index("s") * NC + lax.axis_index("c")
    abase = wid * BA

    pltpu.sync_copy(anc.at[pl.ds(abase, BA)], idx_a)
    pltpu.sync_copy(pos.at[pl.ds(abase, BA)], idx_p)
    pltpu.sync_copy(neg.at[:, pl.ds(abase, CH)], idx_n)

    def idx_src(t):
        if isinstance(t, int) and t == 0:
            return idx_a
        if isinstance(t, int) and t == 1:
            return idx_p
        return idx_n.at[t - 2]

    def dst(t):
        if isinstance(t, int) and t == 0:
            return outa.at[pl.ds(abase, BA)]
        if isinstance(t, int) and t == 1:
            return outp.at[pl.ds(abase, BA)]
        return outn.at[pl.ds((t - 2) * B + abase, CH)]

    def gath(t, j):
        pltpu.async_copy(tab.at[idx_src(t)], bufs[j], gs[j])

    def wait_g(t, j):
        pltpu.make_async_copy(tab.at[idx_src(t)], bufs[j], gs[j]).wait()

    def scat(t, j):
        pltpu.async_copy(bufs[j], dst(t), ss[j])

    def wait_s(t, j):
        pltpu.make_async_copy(bufs[j], dst(t), ss[j]).wait()

    def step(t, j, do_gath):
        if do_gath:
            j2 = (t + DEPTH) % NBUF
            if t + DEPTH - NBUF >= 0:
                wait_s(t + DEPTH - NBUF, j2)
            gath(t + DEPTH, j2)
        wait_g(t, j)
        scat(t, j)

    for t in range(DEPTH):
        gath(t, t % NBUF)

    t0 = max(NBUF - DEPTH, 2)
    n_iter = (T - DEPTH - t0) // NBUF
    t_mid_end = t0 + n_iter * NBUF
    for t in range(t0):
        step(t, t % NBUF, do_gath=True)

    def body(i, _):
        base = t0 + i * NBUF
        for k in range(NBUF):
            t = base + k
            j = (t0 + k) % NBUF
            j2 = (t0 + k + DEPTH) % NBUF
            tw = lax.max(t + DEPTH - NBUF, 2)
            wait_s(tw, j2)
            gath(t + DEPTH, j2)
            wait_g(t, j)
            scat(t, j)
        return _

    lax.fori_loop(0, n_iter, body, None)

    for t in range(t_mid_end, T):
        step(t, t % NBUF, do_gath=(t + DEPTH <= T - 1))

    for t in range(T - NBUF, T):
        wait_s(t, t % NBUF)


def kernel(table, anchor, positive, negative):
    anc = anchor.astype(jnp.int32)
    pos = positive.astype(jnp.int32)
    neg = negative.astype(jnp.int32).T
    outa, outp, outn = _gather3(table, anc, pos, neg)
    return outa, outp, outn.reshape(NNEG, B, D).transpose(1, 0, 2)


# speedup vs baseline: 1.0224x; 1.0224x over previous
import functools

import jax
import jax.numpy as jnp
from jax import lax
from jax.experimental import pallas as pl
from jax.experimental.pallas import tpu as pltpu
from jax.experimental.pallas import tpu_sc as plsc

VOCAB = 100000
D = 128
B = 4096
NNEG = 50

NC = 2
NS = 16
NW = NC * NS

BA = B // NW
CH = BA
NBUF = 6
DEPTH = 3

NCH = NNEG
T = 2 + NCH

_mesh = plsc.VectorSubcoreMesh(core_axis_name="c", subcore_axis_name="s")


@functools.partial(
    pl.kernel,
    out_type=(
        jax.ShapeDtypeStruct((B, D), jnp.float32),
        jax.ShapeDtypeStruct((B, D), jnp.float32),
        jax.ShapeDtypeStruct((B * NNEG, D), jnp.float32),
    ),
    mesh=_mesh,
    scratch_types=(
        [pltpu.VMEM((BA,), jnp.int32),
         pltpu.VMEM((BA,), jnp.int32),
         pltpu.VMEM((NNEG, CH), jnp.int32)]
        + [pltpu.VMEM((CH, D), jnp.float32)] * NBUF
        + [pltpu.SemaphoreType.DMA] * (2 * NBUF + 3)
    ),
)
def _gather3(tab, anc, pos, neg, outa, outp, outn, idx_a, idx_p, idx_n, *rest):
    bufs = rest[:NBUF]
    gs = rest[NBUF:2 * NBUF]
    ss = rest[2 * NBUF:3 * NBUF]
    isems = rest[3 * NBUF:]

    wid = lax.axis_index("s") * NC + lax.axis_index("c")
    abase = wid * BA

    ia = pltpu.async_copy(anc.at[pl.ds(abase, BA)], idx_a, isems[0])
    ip = pltpu.async_copy(pos.at[pl.ds(abase, BA)], idx_p, isems[1])
    inn = pltpu.async_copy(neg.at[:, pl.ds(abase, CH)], idx_n, isems[2])

    def idx_src(t):
        if isinstance(t, int) and t == 0:
            return idx_a
        if isinstance(t, int) and t == 1:
            return idx_p
        return idx_n.at[t - 2]

    def dst(t):
        if isinstance(t, int) and t == 0:
            return outa.at[pl.ds(abase, BA)]
        if isinstance(t, int) and t == 1:
            return outp.at[pl.ds(abase, BA)]
        return outn.at[pl.ds((t - 2) * B + abase, CH)]

    def gath(t, j):
        pltpu.async_copy(tab.at[idx_src(t)], bufs[j], gs[j])

    def wait_g(t, j):
        pltpu.make_async_copy(tab.at[idx_src(t)], bufs[j], gs[j]).wait()

    def scat(t, j):
        pltpu.async_copy(bufs[j], dst(t), ss[j])

    def wait_s(t, j):
        pltpu.make_async_copy(bufs[j], dst(t), ss[j]).wait()

    def step(t, j, do_gath):
        if do_gath:
            j2 = (t + DEPTH) % NBUF
            if t + DEPTH - NBUF >= 0:
                wait_s(t + DEPTH - NBUF, j2)
            gath(t + DEPTH, j2)
        wait_g(t, j)
        scat(t, j)

    ia.wait()
    gath(0, 0 % NBUF)
    ip.wait()
    gath(1, 1 % NBUF)
    inn.wait()
    for t in range(2, DEPTH):
        gath(t, t % NBUF)

    t0 = max(NBUF - DEPTH, 2)
    n_iter = (T - DEPTH - t0) // NBUF
    t_mid_end = t0 + n_iter * NBUF
    for t in range(t0):
        step(t, t % NBUF, do_gath=True)

    def body(i, _):
        base = t0 + i * NBUF
        for k in range(NBUF):
            t = base + k
            j = (t0 + k) % NBUF
            j2 = (t0 + k + DEPTH) % NBUF
            tw = lax.max(t + DEPTH - NBUF, 2)
            wait_s(tw, j2)
            gath(t + DEPTH, j2)
            wait_g(t, j)
            scat(t, j)
        return _

    lax.fori_loop(0, n_iter, body, None)

    for t in range(t_mid_end, T):
        step(t, t % NBUF, do_gath=(t + DEPTH <= T - 1))

    for t in range(T - NBUF, T):
        wait_s(t, t % NBUF)


def kernel(table, anchor, positive, negative):
    anc = anchor.astype(jnp.int32)
    pos = positive.astype(jnp.int32)
    neg = negative.astype(jnp.int32).T
    outa, outp, outn = _gather3(table, anc, pos, neg)
    return outa, outp, outn.reshape(NNEG, B, D).transpose(1, 0, 2)


# speedup vs baseline: 1.0235x; 1.0011x over previous
import functools

import jax
import jax.numpy as jnp
from jax import lax
from jax.experimental import pallas as pl
from jax.experimental.pallas import tpu as pltpu
from jax.experimental.pallas import tpu_sc as plsc

VOCAB = 100000
D = 128
B = 4096
NNEG = 50

NC = 2
NS = 16
NW = NC * NS

BA = B // NW
CH = BA
NBUF = 6
DEPTH = 4

NCH = NNEG
T = 2 + NCH

_mesh = plsc.VectorSubcoreMesh(core_axis_name="c", subcore_axis_name="s")


@functools.partial(
    pl.kernel,
    out_type=(
        jax.ShapeDtypeStruct((B, D), jnp.float32),
        jax.ShapeDtypeStruct((B, D), jnp.float32),
        jax.ShapeDtypeStruct((B * NNEG, D), jnp.float32),
    ),
    mesh=_mesh,
    scratch_types=(
        [pltpu.VMEM((BA,), jnp.int32),
         pltpu.VMEM((BA,), jnp.int32),
         pltpu.VMEM((NNEG, CH), jnp.int32)]
        + [pltpu.VMEM((CH, D), jnp.float32)] * NBUF
        + [pltpu.SemaphoreType.DMA] * (2 * NBUF + 3)
    ),
)
def _gather3(tab, anc, pos, neg, outa, outp, outn, idx_a, idx_p, idx_n, *rest):
    bufs = rest[:NBUF]
    gs = rest[NBUF:2 * NBUF]
    ss = rest[2 * NBUF:3 * NBUF]
    isems = rest[3 * NBUF:]

    wid = lax.axis_index("s") * NC + lax.axis_index("c")
    abase = wid * BA

    ia = pltpu.async_copy(anc.at[pl.ds(abase, BA)], idx_a, isems[0])
    ip = pltpu.async_copy(pos.at[pl.ds(abase, BA)], idx_p, isems[1])
    inn = pltpu.async_copy(neg.at[:, pl.ds(abase, CH)], idx_n, isems[2])

    def idx_src(t):
        if isinstance(t, int) and t == 0:
            return idx_a
        if isinstance(t, int) and t == 1:
            return idx_p
        return idx_n.at[t - 2]

    def dst(t):
        if isinstance(t, int) and t == 0:
            return outa.at[pl.ds(abase, BA)]
        if isinstance(t, int) and t == 1:
            return outp.at[pl.ds(abase, BA)]
        return outn.at[pl.ds((t - 2) * B + abase, CH)]

    def gath(t, j):
        pltpu.async_copy(tab.at[idx_src(t)], bufs[j], gs[j])

    def wait_g(t, j):
        pltpu.make_async_copy(tab.at[idx_src(t)], bufs[j], gs[j]).wait()

    def scat(t, j):
        pltpu.async_copy(bufs[j], dst(t), ss[j])

    def wait_s(t, j):
        pltpu.make_async_copy(bufs[j], dst(t), ss[j]).wait()

    def step(t, j, do_gath):
        if do_gath:
            j2 = (t + DEPTH) % NBUF
            if t + DEPTH - NBUF >= 0:
                wait_s(t + DEPTH - NBUF, j2)
            gath(t + DEPTH, j2)
        wait_g(t, j)
        scat(t, j)

    ia.wait()
    gath(0, 0 % NBUF)
    ip.wait()
    gath(1, 1 % NBUF)
    inn.wait()
    for t in range(2, DEPTH):
        gath(t, t % NBUF)

    t0 = max(NBUF - DEPTH, 2)
    n_iter = (T - DEPTH - t0) // NBUF
    t_mid_end = t0 + n_iter * NBUF
    for t in range(t0):
        step(t, t % NBUF, do_gath=True)

    def body(i, _):
        base = t0 + i * NBUF
        for k in range(NBUF):
            t = base + k
            j = (t0 + k) % NBUF
            j2 = (t0 + k + DEPTH) % NBUF
            tw = lax.max(t + DEPTH - NBUF, 2)
            wait_s(tw, j2)
            gath(t + DEPTH, j2)
            wait_g(t, j)
            scat(t, j)
        return _

    lax.fori_loop(0, n_iter, body, None)

    for t in range(t_mid_end, T):
        step(t, t % NBUF, do_gath=(t + DEPTH <= T - 1))

    for t in range(T - NBUF, T):
        wait_s(t, t % NBUF)


def kernel(table, anchor, positive, negative):
    anc = anchor.astype(jnp.int32)
    pos = positive.astype(jnp.int32)
    neg = negative.astype(jnp.int32).T
    outa, outp, outn = _gather3(table, anc, pos, neg)
    return outa, outp, outn.reshape(NNEG, B, D).transpose(1, 0, 2)
